# baseline (speedup 1.0000x reference)
import functools

import jax
import jax.numpy as jnp
from jax import lax
from jax.experimental import pallas as pl
from jax.experimental.pallas import tpu as pltpu
from jax.experimental.pallas import tpu_sc as plsc

N_NODES = 100000
EMB_DIM = 50
EPS = 1e-7
MIN_NORM = 1e-15
D_PAD = 64
DH = 32
ROW_BLK = 1000

HALF_N = N_NODES // 2
OROWS = 3128
CORE_ROWS = 16 * OROWS
N_OUT = 2 * CORE_ROWS
GAP = CORE_ROWS - HALF_N
DUMP = CORE_ROWS
ZROWS = 3136
ACC_ROWS = 16 * ZROWS

BLK = 2048
SUB = 128
NBLK = 49
PER_TILE = BLK * NBLK
PAD_E = 16 * PER_TILE


def _pre_body(w_ref, o_ref):
    w = w_ref[:, 1:EMB_DIM]
    s = jnp.sum(w * w, axis=1, keepdims=True)
    x0 = jnp.sqrt(jnp.clip(1.0 + s, EPS, None))
    ynorm = jnp.clip(jnp.sqrt(s), MIN_NORM, None)
    theta = jnp.clip(x0, 1.0 + EPS, None)
    acosh = jnp.log(theta + jnp.sqrt(theta * theta - 1.0))
    rest = acosh * w / ynorm
    blk = o_ref.shape[0]
    zeros1 = jnp.zeros((blk, 1), jnp.float32)
    zpad = jnp.zeros((blk, D_PAD - EMB_DIM), jnp.float32)
    o_ref[:, :] = jnp.concatenate([zeros1, rest, zpad], axis=1)


def _pre_map(weight):
    return pl.pallas_call(
        _pre_body,
        grid=(N_NODES // ROW_BLK,),
        in_specs=[pl.BlockSpec((ROW_BLK, EMB_DIM), lambda i: (i, 0))],
        out_specs=pl.BlockSpec((ROW_BLK, D_PAD), lambda i: (i, 0)),
        out_shape=jax.ShapeDtypeStruct((N_NODES, D_PAD), jnp.float32),
    )(weight)


def _post_body(a_ref, b_ref, c_ref, o_ref):
    u = a_ref[:, :] + b_ref[:, :] + c_ref[:, :]
    x = u[:, 1:EMB_DIM]
    xn = jnp.clip(jnp.sqrt(jnp.sum(x * x, axis=1, keepdims=True)), MIN_NORM, None)
    e = jnp.exp(xn)
    sh = 0.5 * (e - 1.0 / e)
    rest = sh * x / xn
    s2 = jnp.sum(rest * rest, axis=1, keepdims=True)
    x0 = jnp.sqrt(jnp.clip(1.0 + s2, EPS, None))
    o_ref[:, :] = jnp.concatenate([x0, rest], axis=1)


def _post_map(y1, y2, y3):
    spec = pl.BlockSpec((ROW_BLK, D_PAD), lambda i: (i, 0))
    return pl.pallas_call(
        _post_body,
        grid=(N_NODES // ROW_BLK,),
        in_specs=[spec, spec, spec],
        out_specs=pl.BlockSpec((ROW_BLK, EMB_DIM), lambda i: (i, 0)),
        out_shape=jax.ShapeDtypeStruct((N_NODES, EMB_DIM), jnp.float32),
    )(y1, y2, y3)


def _spmm_body(xh0, xh1, src_h, dst_h, w_h, zeros_h, out0, out1,
               dst_b, src_b, w_b, dstl, srcs, ws, rows, acc, sem):
    c = lax.axis_index("c")
    s = lax.axis_index("s")
    lo = c * HALF_N
    obase = c * CORE_ROWS
    ebase = s * PER_TILE

    for h in range(2):
        xh = (xh0, xh1)[h]
        outh = (out0, out1)[h]

        pltpu.sync_copy(zeros_h, acc.at[pl.ds(s * ZROWS, ZROWS)])
        plsc.subcore_barrier()

        def blk_body(b, _):
            off = ebase + b * BLK
            pltpu.sync_copy(dst_h.at[pl.ds(off, BLK)], dst_b)
            pltpu.sync_copy(src_h.at[pl.ds(off, BLK)], src_b)
            pltpu.sync_copy(w_h.at[pl.ds(off, BLK)], w_b)

            def sub_body(sb, _):
                def grp_body(g, _):
                    o = sb * SUB + g * 16
                    dv = dst_b[pl.ds(o, 16)]
                    sv = src_b[pl.ds(o, 16)]
                    wv = w_b[pl.ds(o, 16)]
                    m = (dv >= lo) & (dv < lo + HALF_N)
                    dstl[pl.ds(g * 16, 16)] = jnp.where(m, dv - lo, DUMP)
                    srcs[pl.ds(g * 16, 16)] = jnp.where(m, sv, 0)
                    ws[pl.ds(g * 16, 16)] = jnp.where(m, wv, 0.0)
                    return 0

                lax.fori_loop(0, SUB // 16, grp_body, 0)
                pltpu.async_copy(xh.at[srcs], rows, sem).wait()

                def row_body(g, _):
                    w16 = ws[pl.ds(g * 16, 16)]
                    for j in range(16):
                        i = g * 16 + j
                        wbc = jnp.full((16,), w16[j], jnp.float32)
                        a = rows[i, pl.ds(0, 16)]
                        rows[i, pl.ds(0, 16)] = a * wbc
                        b2 = rows[i, pl.ds(16, 16)]
                        rows[i, pl.ds(16, 16)] = b2 * wbc
                    return 0

                lax.fori_loop(0, SUB // 16, row_body, 0)
                pltpu.sync_copy(rows, acc.at[dstl], add=True)
                return 0

            lax.fori_loop(0, BLK // SUB, sub_body, 0)
            return 0

        lax.fori_loop(0, NBLK, blk_body, 0)
        plsc.subcore_barrier()
        pltpu.sync_copy(acc.at[pl.ds(s * OROWS, OROWS)],
                        outh.at[pl.ds(obase + s * OROWS, OROWS)])
        plsc.subcore_barrier()


_spmm_call = functools.partial(
    pl.kernel,
    mesh=plsc.VectorSubcoreMesh(core_axis_name="c", subcore_axis_name="s"),
    compiler_params=pltpu.CompilerParams(use_tc_tiling_on_sc=False),
    out_type=[
        jax.ShapeDtypeStruct((N_OUT, DH), jnp.float32),
        jax.ShapeDtypeStruct((N_OUT, DH), jnp.float32),
    ],
    scratch_types=[
        pltpu.VMEM((BLK,), jnp.int32),
        pltpu.VMEM((BLK,), jnp.int32),
        pltpu.VMEM((BLK,), jnp.float32),
        pltpu.VMEM((SUB,), jnp.int32),
        pltpu.VMEM((SUB,), jnp.int32),
        pltpu.VMEM((SUB,), jnp.float32),
        pltpu.VMEM((SUB, DH), jnp.float32),
        pltpu.VMEM_SHARED((ACC_ROWS, DH), jnp.float32),
        pltpu.SemaphoreType.DMA,
    ],
)(_spmm_body)


def _degap(h0, h1):
    full = jnp.concatenate([h0, h1], axis=1)
    return jnp.concatenate(
        [full[:HALF_N], full[CORE_ROWS:CORE_ROWS + HALF_N]], axis=0)


def kernel(weight, edge_index, edge_weight):
    src = edge_index[0].astype(jnp.int32)
    dst = edge_index[1].astype(jnp.int32)
    src = src + jnp.where(src >= HALF_N, GAP, 0).astype(jnp.int32)
    pad = PAD_E - src.shape[0]
    src = jnp.pad(src, (0, pad))
    dst = jnp.pad(dst, (0, pad))
    w = jnp.pad(edge_weight, (0, pad))
    zeros = jnp.zeros((ZROWS, DH), jnp.float32)

    xt = _pre_map(weight)
    xtp = jnp.concatenate(
        [xt[:HALF_N], jnp.zeros((GAP, D_PAD), jnp.float32), xt[HALF_N:],
         jnp.zeros((GAP, D_PAD), jnp.float32)], axis=0)
    h0, h1 = xtp[:, :DH], xtp[:, DH:]
    y1h0, y1h1 = _spmm_call(h0, h1, src, dst, w, zeros)
    y2h0, y2h1 = _spmm_call(y1h0, y1h1, src, dst, w, zeros)
    y3h0, y3h1 = _spmm_call(y2h0, y2h1, src, dst, w, zeros)
    return _post_map(_degap(y1h0, y1h1), _degap(y2h0, y2h1),
                     _degap(y3h0, y3h1))
